# Initial kernel scaffold; baseline (speedup 1.0000x reference)
import jax
import jax.numpy as jnp
from jax.experimental import pallas as pl


def kernel(batch, label, table):
    raise NotImplementedError("write your pallas kernel here")


# speedup vs baseline: 1.8861x; 1.8861x over previous
import functools

import jax
import jax.numpy as jnp
from jax import lax
from jax.experimental import pallas as pl
from jax.experimental.pallas import tpu as pltpu
from jax.experimental.pallas import tpu_sc as plsc

_NUM_CORES = 2
_NUM_SUBCORES = 16
_NUM_WORKERS = _NUM_CORES * _NUM_SUBCORES
_LANES = 16
_TABLE_PAD = 64


@functools.lru_cache(maxsize=None)
def _make_lookup(batch_size: int, vocab: int, dim: int):
    assert batch_size % (_NUM_WORKERS * _LANES) == 0
    assert vocab * dim <= _TABLE_PAD
    assert dim & (dim - 1) == 0 and dim <= _LANES
    b_per_w = batch_size // _NUM_WORKERS
    out_per_w = b_per_w * dim
    n_groups = out_per_w // _LANES

    mesh = plsc.VectorSubcoreMesh(core_axis_name="c", subcore_axis_name="s")

    @functools.partial(
        pl.kernel,
        mesh=mesh,
        out_type=jax.ShapeDtypeStruct((_NUM_WORKERS, out_per_w), jnp.float32),
        scratch_types=[
            pltpu.VMEM((b_per_w,), jnp.int32),
            pltpu.VMEM((_TABLE_PAD,), jnp.float32),
            pltpu.VMEM((out_per_w,), jnp.float32),
        ],
        compiler_params=pltpu.CompilerParams(needs_layout_passes=False),
    )
    def lookup(label_hbm, table_hbm, out_hbm, idx_v, table_v, out_v):
        wid = lax.axis_index("s") * _NUM_CORES + lax.axis_index("c")
        pltpu.sync_copy(table_hbm, table_v)
        pltpu.sync_copy(label_hbm.at[wid], idx_v)
        lane = lax.iota(jnp.int32, _LANES)
        shift = dim.bit_length() - 1
        row0 = lax.shift_right_logical(lane, shift)
        col = lax.bitwise_and(lane, dim - 1)
        rows_per_group = _LANES // dim
        for g in range(n_groups):
            lab = plsc.load_gather(idx_v, [lax.add(row0, g * rows_per_group)])
            val = plsc.load_gather(
                table_v, [lax.bitwise_or(lax.shift_left(lab, shift), col)]
            )
            out_v[pl.ds(g * _LANES, _LANES)] = val
        pltpu.sync_copy(out_v, out_hbm.at[wid])

    return lookup


def kernel(batch, label, table):
    del batch
    (batch_size,) = label.shape
    vocab, dim = table.shape
    label_r = label.reshape(_NUM_WORKERS, batch_size // _NUM_WORKERS)
    table_flat = jnp.pad(table.reshape(-1), (0, _TABLE_PAD - vocab * dim))
    out = _make_lookup(batch_size, vocab, dim)(label_r, table_flat)
    return out.reshape(batch_size, dim)


# speedup vs baseline: 2.1556x; 1.1429x over previous
import functools

import jax
import jax.numpy as jnp
from jax import lax
from jax.experimental import pallas as pl
from jax.experimental.pallas import tpu as pltpu
from jax.experimental.pallas import tpu_sc as plsc

_NUM_CORES = 2
_NUM_SUBCORES = 16
_NUM_WORKERS = _NUM_CORES * _NUM_SUBCORES
_LANES = 16
_TABLE_PAD = 64


@functools.lru_cache(maxsize=None)
def _make_lookup(batch_size: int, vocab: int, dim: int):
    assert batch_size % (_NUM_WORKERS * _LANES) == 0
    assert vocab * dim <= _TABLE_PAD
    assert dim & (dim - 1) == 0 and dim <= _LANES
    b_per_w = batch_size // _NUM_WORKERS
    out_per_w = b_per_w * dim
    n_groups = out_per_w // _LANES

    mesh = plsc.VectorSubcoreMesh(core_axis_name="c", subcore_axis_name="s")

    @functools.partial(
        pl.kernel,
        mesh=mesh,
        out_type=jax.ShapeDtypeStruct((_NUM_WORKERS, out_per_w), jnp.float32),
        scratch_types=[
            pltpu.VMEM((b_per_w,), jnp.int32),
            pltpu.VMEM((_TABLE_PAD,), jnp.float32),
            pltpu.VMEM((out_per_w,), jnp.float32),
        ],
        compiler_params=pltpu.CompilerParams(needs_layout_passes=False),
    )
    def lookup(label_hbm, table_hbm, out_hbm, idx_v, table_v, out_v):
        wid = lax.axis_index("s") * _NUM_CORES + lax.axis_index("c")
        pltpu.sync_copy(table_hbm, table_v)
        pltpu.sync_copy(label_hbm.at[wid], idx_v)
        lane = lax.iota(jnp.int32, _LANES)
        shift = dim.bit_length() - 1
        row0 = lax.shift_right_logical(lane, shift)
        col = lax.bitwise_and(lane, dim - 1)
        rows_per_group = _LANES // dim

        @plsc.parallel_loop(0, n_groups, 1, unroll=8)
        def _group(g):
            lab = plsc.load_gather(idx_v, [row0 + g * rows_per_group])
            val = plsc.load_gather(
                table_v, [lax.bitwise_or(lax.shift_left(lab, shift), col)]
            )
            out_v[pl.ds(g * _LANES, _LANES)] = val
        pltpu.sync_copy(out_v, out_hbm.at[wid])

    return lookup


def kernel(batch, label, table):
    del batch
    (batch_size,) = label.shape
    vocab, dim = table.shape
    label_r = label.reshape(_NUM_WORKERS, batch_size // _NUM_WORKERS)
    table_flat = jnp.pad(table.reshape(-1), (0, _TABLE_PAD - vocab * dim))
    out = _make_lookup(batch_size, vocab, dim)(label_r, table_flat)
    return out.reshape(batch_size, dim)


# speedup vs baseline: 2.5402x; 1.1784x over previous
import functools

import jax
import jax.numpy as jnp
from jax import lax
from jax.experimental import pallas as pl
from jax.experimental.pallas import tpu as pltpu
from jax.experimental.pallas import tpu_sc as plsc

_NUM_CORES = 2
_NUM_SUBCORES = 16
_NUM_WORKERS = _NUM_CORES * _NUM_SUBCORES
_LANES = 16


@functools.lru_cache(maxsize=None)
def _make_lookup(batch_size: int, vocab: int, dim: int):
    assert batch_size % (_NUM_WORKERS * _LANES) == 0
    assert dim & (dim - 1) == 0 and dim <= _LANES
    b_per_w = batch_size // _NUM_WORKERS
    out_per_w = b_per_w * dim
    n_groups = out_per_w // _LANES

    mesh = plsc.VectorSubcoreMesh(core_axis_name="c", subcore_axis_name="s")

    @functools.partial(
        pl.kernel,
        mesh=mesh,
        out_type=jax.ShapeDtypeStruct((batch_size, dim), jnp.float32),
        scratch_types=[
            pltpu.VMEM((b_per_w,), jnp.int32),
            pltpu.VMEM((vocab, dim), jnp.float32),
            pltpu.VMEM((b_per_w, dim), jnp.float32),
        ],
        compiler_params=pltpu.CompilerParams(needs_layout_passes=False),
    )
    def lookup(label_hbm, table_hbm, out_hbm, idx_v, table_v, out_v):
        wid = lax.axis_index("s") * _NUM_CORES + lax.axis_index("c")
        base = wid * b_per_w
        pltpu.sync_copy(table_hbm, table_v)
        pltpu.sync_copy(label_hbm.at[pl.ds(base, b_per_w)], idx_v)
        lane = lax.iota(jnp.int32, _LANES)
        shift = dim.bit_length() - 1
        row0 = lax.shift_right_logical(lane, shift)
        col = lax.bitwise_and(lane, dim - 1)
        rows_per_group = _LANES // dim

        @plsc.parallel_loop(0, n_groups, 1, unroll=8)
        def _group(g):
            row = row0 + g * rows_per_group
            lab = plsc.load_gather(idx_v, [row])
            val = plsc.load_gather(table_v, [lab, col])
            plsc.store_scatter(out_v, [row, col], val)

        pltpu.sync_copy(out_v, out_hbm.at[pl.ds(base, b_per_w)])

    return lookup


def kernel(batch, label, table):
    del batch
    (batch_size,) = label.shape
    vocab, dim = table.shape
    return _make_lookup(batch_size, vocab, dim)(label, table)
